# Initial kernel scaffold; baseline (speedup 1.0000x reference)
import jax
import jax.numpy as jnp
from jax.experimental import pallas as pl


def kernel(x, edge_index, W1, b1, W2, b2, W3, b3):
    raise NotImplementedError("write your pallas kernel here")


# speedup vs baseline: 3.8727x; 3.8727x over previous
import functools

import jax
import jax.numpy as jnp
from jax import lax
from jax.experimental import pallas as pl
from jax.experimental.pallas import tpu as pltpu
from jax.experimental.pallas import tpu_sc as plsc

N = 10000
NP = 10240
E = 160000
EP = 163840
NB = 40
B = 128
NC = 2
NS = 16
ROWS = NP // NS
MB = 512


def _zero_rows(zbuf, fc):
    @pl.loop(0, B)
    def _(i):
        for k in range(fc // 16):
            zbuf[i, pl.ds(k * 16, 16)] = jnp.zeros((16,), jnp.float32)


def _make_deg():
    mesh = plsc.VectorSubcoreMesh(core_axis_name="c", subcore_axis_name="s")

    @functools.partial(
        pl.kernel,
        out_type=jax.ShapeDtypeStruct((NC, 2, NP, 128), jnp.float32),
        mesh=mesh,
        scratch_types=[
            pltpu.VMEM((NB, B), jnp.int32),
            pltpu.VMEM((NB, B), jnp.int32),
            pltpu.VMEM((B, 128), jnp.float32),
            pltpu.VMEM((B, 128), jnp.float32),
            pltpu.VMEM_SHARED((NP, 128), jnp.float32),
        ],
    )
    def deg(src_hbm, dst_hbm, out_hbm, idx_s, idx_d, obuf, zbuf, acc):
        c = lax.axis_index("c")
        s = lax.axis_index("s")
        tid = c * NS + s
        pltpu.sync_copy(src_hbm.at[tid], idx_s)
        pltpu.sync_copy(dst_hbm.at[tid], idx_d)

        @pl.loop(0, B)
        def _(i):
            for k in range(8):
                obuf[i, pl.ds(k * 16, 16)] = jnp.ones((16,), jnp.float32)
                zbuf[i, pl.ds(k * 16, 16)] = jnp.zeros((16,), jnp.float32)

        row0 = s * ROWS
        for kind, idx in ((0, idx_s), (1, idx_d)):
            for r in range(ROWS // B):
                pltpu.sync_copy(zbuf, acc.at[pl.ds(row0 + r * B, B)])
            plsc.subcore_barrier()

            @pl.loop(0, NB)
            def _(j):
                pltpu.sync_copy(obuf, acc.at[idx.at[j]], add=True)

            plsc.subcore_barrier()
            pltpu.sync_copy(acc.at[pl.ds(row0, ROWS)],
                            out_hbm.at[c, kind, pl.ds(row0, ROWS)])
            if kind == 0:
                plsc.subcore_barrier()

    return deg


def _make_agg(C, FC):
    mesh = plsc.VectorSubcoreMesh(core_axis_name="c", subcore_axis_name="s")

    @functools.partial(
        pl.kernel,
        out_type=jax.ShapeDtypeStruct((NC, C, NP, FC), jnp.float32),
        mesh=mesh,
        scratch_types=[
            pltpu.VMEM((NB, B), jnp.int32),
            pltpu.VMEM((NB, B), jnp.int32),
            pltpu.VMEM((B, FC), jnp.float32),
            pltpu.VMEM((B, FC), jnp.float32),
            pltpu.VMEM_SHARED((NP, FC), jnp.float32),
            pltpu.SemaphoreType.DMA,
        ],
    )
    def agg(y_hbm, src_hbm, dst_hbm, out_hbm, idx_s, idx_d, gbuf, zbuf, acc,
            sem):
        c = lax.axis_index("c")
        s = lax.axis_index("s")
        tid = c * NS + s
        pltpu.sync_copy(src_hbm.at[tid], idx_s)
        pltpu.sync_copy(dst_hbm.at[tid], idx_d)
        _zero_rows(zbuf, FC)

        row0 = s * ROWS
        for ci in range(C):
            for r in range(ROWS // B):
                pltpu.sync_copy(zbuf, acc.at[pl.ds(row0 + r * B, B)])
            plsc.subcore_barrier()

            @pl.loop(0, NB)
            def _(j):
                pltpu.async_copy(y_hbm.at[ci].at[idx_s.at[j]], gbuf,
                                 sem).wait()
                pltpu.sync_copy(gbuf, acc.at[idx_d.at[j]], add=True)

            plsc.subcore_barrier()
            pltpu.sync_copy(acc.at[pl.ds(row0, ROWS)],
                            out_hbm.at[c, ci, pl.ds(row0, ROWS)])
            if ci + 1 < C:
                plsc.subcore_barrier()

    return agg


def _rs(deg_ref, kind):
    d = deg_ref[0, kind, :, 0:1] + deg_ref[1, kind, :, 0:1]
    return lax.rsqrt(jnp.maximum(d, 1.0))


def _mm1_body(x_ref, deg_ref, w_ref, o_ref):
    rout = _rs(deg_ref, 0)
    o_ref[0] = jnp.dot(x_ref[...] * rout, w_ref[...],
                       preferred_element_type=jnp.float32)


def _mm_mid_body(p_ref, deg_ref, b_ref, w_ref, o_ref, *, cin, fout):
    rin = _rs(deg_ref, 1)
    rout = _rs(deg_ref, 0)
    acc = jnp.zeros((MB, fout), jnp.float32)
    for k in range(cin):
        h = jnp.maximum((p_ref[0, k] + p_ref[1, k]) * rin + b_ref[k], 0.0)
        acc += jnp.dot(h * rout, w_ref[k], preferred_element_type=jnp.float32)
    if len(o_ref.shape) == 3:
        o_ref[0] = acc
    else:
        o_ref[...] = acc


def _fin_body(p_ref, deg_ref, b_ref, o_ref):
    rin = _rs(deg_ref, 1)
    o_ref[...] = (p_ref[0, 0, :, :64] + p_ref[1, 0, :, :64]) * rin + b_ref[0]


def _mm1(xp, deg, W1):
    return pl.pallas_call(
        _mm1_body,
        grid=(NP // MB, 4),
        in_specs=[
            pl.BlockSpec((MB, 256), lambda m, c: (m, 0)),
            pl.BlockSpec((2, 2, MB, 128), lambda m, c: (0, 0, m, 0)),
            pl.BlockSpec((256, 128), lambda m, c: (0, c)),
        ],
        out_specs=pl.BlockSpec((1, MB, 128), lambda m, c: (c, m, 0)),
        out_shape=jax.ShapeDtypeStruct((4, NP, 128), jnp.float32),
    )(xp, deg, W1)


def _mm2(p1, deg, b1r, W2r):
    return pl.pallas_call(
        functools.partial(_mm_mid_body, cin=4, fout=128),
        grid=(NP // MB, 4),
        in_specs=[
            pl.BlockSpec((2, 4, MB, 128), lambda m, c: (0, 0, m, 0)),
            pl.BlockSpec((2, 2, MB, 128), lambda m, c: (0, 0, m, 0)),
            pl.BlockSpec((4, 128), lambda m, c: (0, 0)),
            pl.BlockSpec((4, 128, 128), lambda m, c: (0, 0, c)),
        ],
        out_specs=pl.BlockSpec((1, MB, 128), lambda m, c: (c, m, 0)),
        out_shape=jax.ShapeDtypeStruct((4, NP, 128), jnp.float32),
    )(p1, deg, b1r, W2r)


def _mm3(p2, deg, b2r, W3r):
    return pl.pallas_call(
        functools.partial(_mm_mid_body, cin=4, fout=128),
        grid=(NP // MB,),
        in_specs=[
            pl.BlockSpec((2, 4, MB, 128), lambda m: (0, 0, m, 0)),
            pl.BlockSpec((2, 2, MB, 128), lambda m: (0, 0, m, 0)),
            pl.BlockSpec((4, 128), lambda m: (0, 0)),
            pl.BlockSpec((4, 128, 128), lambda m: (0, 0, 0)),
        ],
        out_specs=pl.BlockSpec((MB, 128), lambda m: (m, 0)),
        out_shape=jax.ShapeDtypeStruct((NP, 128), jnp.float32),
    )(p2, deg, b2r, W3r)


def _fin(p3, deg, b3r):
    return pl.pallas_call(
        _fin_body,
        grid=(NP // MB,),
        in_specs=[
            pl.BlockSpec((2, 1, MB, 128), lambda m: (0, 0, m, 0)),
            pl.BlockSpec((2, 2, MB, 128), lambda m: (0, 0, m, 0)),
            pl.BlockSpec((1, 64), lambda m: (0, 0)),
        ],
        out_specs=pl.BlockSpec((MB, 64), lambda m: (m, 0)),
        out_shape=jax.ShapeDtypeStruct((NP, 64), jnp.float32),
    )(p3, deg, b3r)


_deg_kernel = _make_deg()
_agg4 = _make_agg(4, 128)
_agg1 = _make_agg(1, 128)


def kernel(x, edge_index, W1, b1, W2, b2, W3, b3):
    src = edge_index[0]
    dst = edge_index[1]
    npad = EP - E
    pad_idx = N + (jnp.arange(npad, dtype=jnp.int32) % (NP - N))
    srcp = jnp.concatenate([src, pad_idx]).reshape(NC * NS, NB, B)
    dstp = jnp.concatenate([dst, pad_idx]).reshape(NC * NS, NB, B)
    xp = jnp.pad(x, ((0, NP - N), (0, 0)))

    deg = _deg_kernel(srcp, dstp)
    y1 = _mm1(xp, deg, W1)
    p1 = _agg4(y1, srcp, dstp)
    y2 = _mm2(p1, deg, b1.reshape(4, 128), W2.reshape(4, 128, 512))
    p2 = _agg4(y2, srcp, dstp)
    W3p = jnp.pad(W3.reshape(4, 128, 64), ((0, 0), (0, 0), (0, 64)))
    y3 = _mm3(p2, deg, b2.reshape(4, 128), W3p)
    p3 = _agg1(y3.reshape(1, NP, 128), srcp, dstp)
    out = _fin(p3, deg, b3.reshape(1, 64))
    return out[:N]


# speedup vs baseline: 4.1757x; 1.0783x over previous
import functools

import jax
import jax.numpy as jnp
from jax import lax
from jax.experimental import pallas as pl
from jax.experimental.pallas import tpu as pltpu
from jax.experimental.pallas import tpu_sc as plsc

N = 10000
NP = 10240
E = 160000
EP = 163840
NB = 40
B = 128
NC = 2
NS = 16
ROWS = NP // NS
MB = 512


def _make_deg():
    mesh = plsc.VectorSubcoreMesh(core_axis_name="c", subcore_axis_name="s")

    @functools.partial(
        pl.kernel,
        out_type=jax.ShapeDtypeStruct((NC, 2, NP, 128), jnp.float32),
        mesh=mesh,
        scratch_types=[
            pltpu.VMEM((NB, B), jnp.int32),
            pltpu.VMEM((NB, B), jnp.int32),
            pltpu.VMEM((B, 128), jnp.float32),
            pltpu.VMEM_SHARED((NP, 128), jnp.float32),
            pltpu.SemaphoreType.DMA,
        ],
    )
    def deg(src_hbm, dst_hbm, z_hbm, out_hbm, idx_s, idx_d, obuf, acc, sem):
        c = lax.axis_index("c")
        s = lax.axis_index("s")
        tid = c * NS + s
        pltpu.sync_copy(src_hbm.at[tid], idx_s)
        pltpu.sync_copy(dst_hbm.at[tid], idx_d)

        @pl.loop(0, B)
        def _(i):
            for k in range(8):
                obuf[i, pl.ds(k * 16, 16)] = jnp.ones((16,), jnp.float32)

        row0 = s * ROWS
        for kind, idx in ((0, idx_s), (1, idx_d)):
            pltpu.sync_copy(z_hbm.at[pl.ds(row0, ROWS)],
                            acc.at[pl.ds(row0, ROWS)])
            plsc.subcore_barrier()

            @pl.loop(0, NB // 8)
            def _(g):
                descs = [
                    pltpu.async_copy(obuf, acc.at[idx.at[g * 8 + b]], sem,
                                     add=True)
                    for b in range(8)
                ]
                for d in descs:
                    d.wait()

            plsc.subcore_barrier()
            pltpu.sync_copy(acc.at[pl.ds(row0, ROWS)],
                            out_hbm.at[c, kind, pl.ds(row0, ROWS)])
            if kind == 0:
                plsc.subcore_barrier()

    return deg


K_BUF = 2


def _make_agg(C, FC):
    mesh = plsc.VectorSubcoreMesh(core_axis_name="c", subcore_axis_name="s")

    @functools.partial(
        pl.kernel,
        out_type=jax.ShapeDtypeStruct((NC, C, NP, FC), jnp.float32),
        mesh=mesh,
        scratch_types=(
            [pltpu.VMEM((NB, B), jnp.int32),
             pltpu.VMEM((NB, B), jnp.int32)]
            + [pltpu.VMEM((B, FC), jnp.float32) for _ in range(K_BUF)]
            + [pltpu.VMEM_SHARED((NP, FC), jnp.float32)]
            + [pltpu.SemaphoreType.DMA for _ in range(2 * K_BUF)]
        ),
    )
    def agg(y_hbm, src_hbm, dst_hbm, z_hbm, out_hbm, idx_s, idx_d, *rest):
        gbufs = rest[:K_BUF]
        acc = rest[K_BUF]
        gsems = rest[K_BUF + 1:2 * K_BUF + 1]
        ssems = rest[2 * K_BUF + 1:]
        c = lax.axis_index("c")
        s = lax.axis_index("s")
        tid = c * NS + s
        pltpu.sync_copy(src_hbm.at[tid], idx_s)
        pltpu.sync_copy(dst_hbm.at[tid], idx_d)

        row0 = s * ROWS
        for ci in range(C):
            pltpu.sync_copy(z_hbm.at[pl.ds(row0, ROWS)],
                            acc.at[pl.ds(row0, ROWS)])
            plsc.subcore_barrier()

            @pl.loop(0, NB // K_BUF)
            def _(g):
                j0 = g * K_BUF
                gds = [
                    pltpu.async_copy(y_hbm.at[ci].at[idx_s.at[j0 + b]],
                                     gbufs[b], gsems[b])
                    for b in range(K_BUF)
                ]
                sds = []
                for b in range(K_BUF):
                    gds[b].wait()
                    sds.append(
                        pltpu.async_copy(gbufs[b], acc.at[idx_d.at[j0 + b]],
                                         ssems[b], add=True))
                for d in sds:
                    d.wait()

            plsc.subcore_barrier()
            pltpu.sync_copy(acc.at[pl.ds(row0, ROWS)],
                            out_hbm.at[c, ci, pl.ds(row0, ROWS)])
            if ci + 1 < C:
                plsc.subcore_barrier()

    return agg


def _rs(deg_ref, kind):
    d = deg_ref[0, kind, :, 0:1] + deg_ref[1, kind, :, 0:1]
    return lax.rsqrt(jnp.maximum(d, 1.0))


def _mm1_body(x_ref, deg_ref, w_ref, o_ref):
    rout = _rs(deg_ref, 0)
    o_ref[0] = jnp.dot(x_ref[...] * rout, w_ref[...],
                       preferred_element_type=jnp.float32)


def _mm_mid_body(p_ref, deg_ref, b_ref, w_ref, o_ref, *, cin, fout):
    rin = _rs(deg_ref, 1)
    rout = _rs(deg_ref, 0)
    acc = jnp.zeros((MB, fout), jnp.float32)
    for k in range(cin):
        h = jnp.maximum((p_ref[0, k] + p_ref[1, k]) * rin + b_ref[k], 0.0)
        acc += jnp.dot(h * rout, w_ref[k], preferred_element_type=jnp.float32)
    if len(o_ref.shape) == 3:
        o_ref[0] = acc
    else:
        o_ref[...] = acc


def _fin_body(p_ref, deg_ref, b_ref, o_ref):
    rin = _rs(deg_ref, 1)
    o_ref[...] = (p_ref[0, 0, :, :64] + p_ref[1, 0, :, :64]) * rin + b_ref[0]


def _mm1(xp, deg, W1):
    return pl.pallas_call(
        _mm1_body,
        grid=(NP // MB, 4),
        in_specs=[
            pl.BlockSpec((MB, 256), lambda m, c: (m, 0)),
            pl.BlockSpec((2, 2, MB, 128), lambda m, c: (0, 0, m, 0)),
            pl.BlockSpec((256, 128), lambda m, c: (0, c)),
        ],
        out_specs=pl.BlockSpec((1, MB, 128), lambda m, c: (c, m, 0)),
        out_shape=jax.ShapeDtypeStruct((4, NP, 128), jnp.float32),
    )(xp, deg, W1)


def _mm2(p1, deg, b1r, W2r):
    return pl.pallas_call(
        functools.partial(_mm_mid_body, cin=4, fout=128),
        grid=(NP // MB, 4),
        in_specs=[
            pl.BlockSpec((2, 4, MB, 128), lambda m, c: (0, 0, m, 0)),
            pl.BlockSpec((2, 2, MB, 128), lambda m, c: (0, 0, m, 0)),
            pl.BlockSpec((4, 128), lambda m, c: (0, 0)),
            pl.BlockSpec((4, 128, 128), lambda m, c: (0, 0, c)),
        ],
        out_specs=pl.BlockSpec((1, MB, 128), lambda m, c: (c, m, 0)),
        out_shape=jax.ShapeDtypeStruct((4, NP, 128), jnp.float32),
    )(p1, deg, b1r, W2r)


def _mm3(p2, deg, b2r, W3r):
    return pl.pallas_call(
        functools.partial(_mm_mid_body, cin=4, fout=128),
        grid=(NP // MB,),
        in_specs=[
            pl.BlockSpec((2, 4, MB, 128), lambda m: (0, 0, m, 0)),
            pl.BlockSpec((2, 2, MB, 128), lambda m: (0, 0, m, 0)),
            pl.BlockSpec((4, 128), lambda m: (0, 0)),
            pl.BlockSpec((4, 128, 128), lambda m: (0, 0, 0)),
        ],
        out_specs=pl.BlockSpec((MB, 128), lambda m: (m, 0)),
        out_shape=jax.ShapeDtypeStruct((NP, 128), jnp.float32),
    )(p2, deg, b2r, W3r)


def _fin(p3, deg, b3r):
    return pl.pallas_call(
        _fin_body,
        grid=(NP // MB,),
        in_specs=[
            pl.BlockSpec((2, 1, MB, 128), lambda m: (0, 0, m, 0)),
            pl.BlockSpec((2, 2, MB, 128), lambda m: (0, 0, m, 0)),
            pl.BlockSpec((1, 64), lambda m: (0, 0)),
        ],
        out_specs=pl.BlockSpec((MB, 64), lambda m: (m, 0)),
        out_shape=jax.ShapeDtypeStruct((NP, 64), jnp.float32),
    )(p3, deg, b3r)


_deg_kernel = _make_deg()
_agg4 = _make_agg(4, 128)
_agg1 = _make_agg(1, 128)


def kernel(x, edge_index, W1, b1, W2, b2, W3, b3):
    src = edge_index[0]
    dst = edge_index[1]
    npad = EP - E
    pad_idx = N + (jnp.arange(npad, dtype=jnp.int32) % (NP - N))
    srcp = jnp.concatenate([src, pad_idx]).reshape(NC * NS, NB, B)
    dstp = jnp.concatenate([dst, pad_idx]).reshape(NC * NS, NB, B)
    xp = jnp.pad(x, ((0, NP - N), (0, 0)))

    zeros = jnp.zeros((NP, 128), jnp.float32)
    deg = _deg_kernel(srcp, dstp, zeros)
    y1 = _mm1(xp, deg, W1)
    p1 = _agg4(y1, srcp, dstp, zeros)
    y2 = _mm2(p1, deg, b1.reshape(4, 128), W2.reshape(4, 128, 512))
    p2 = _agg4(y2, srcp, dstp, zeros)
    W3p = jnp.pad(W3.reshape(4, 128, 64), ((0, 0), (0, 0), (0, 64)))
    y3 = _mm3(p2, deg, b2.reshape(4, 128), W3p)
    p3 = _agg1(y3.reshape(1, NP, 128), srcp, dstp, zeros)
    out = _fin(p3, deg, b3.reshape(1, 64))
    return out[:N]


# speedup vs baseline: 4.7285x; 1.1324x over previous
import functools

import jax
import jax.numpy as jnp
from jax import lax
from jax.experimental import pallas as pl
from jax.experimental.pallas import tpu as pltpu
from jax.experimental.pallas import tpu_sc as plsc

N = 10000
NP = 10240
E = 160000
EP = 163840
NB = 40
B = 128
NC = 2
NS = 16
ROWS = NP // NS
MB = 512


def _make_deg():
    mesh = plsc.VectorSubcoreMesh(core_axis_name="c", subcore_axis_name="s")

    @functools.partial(
        pl.kernel,
        out_type=jax.ShapeDtypeStruct((NC, 2, NP, 128), jnp.float32),
        mesh=mesh,
        scratch_types=[
            pltpu.VMEM((NB, B), jnp.int32),
            pltpu.VMEM((NB, B), jnp.int32),
            pltpu.VMEM((B, 128), jnp.float32),
            pltpu.VMEM_SHARED((NP, 128), jnp.float32),
            pltpu.SemaphoreType.DMA,
        ],
    )
    def deg(src_hbm, dst_hbm, z_hbm, out_hbm, idx_s, idx_d, obuf, acc, sem):
        c = lax.axis_index("c")
        s = lax.axis_index("s")
        tid = c * NS + s
        pltpu.sync_copy(src_hbm.at[tid], idx_s)
        pltpu.sync_copy(dst_hbm.at[tid], idx_d)

        @pl.loop(0, B)
        def _(i):
            for k in range(8):
                obuf[i, pl.ds(k * 16, 16)] = jnp.ones((16,), jnp.float32)

        row0 = s * ROWS
        for kind, idx in ((0, idx_s), (1, idx_d)):
            pltpu.sync_copy(z_hbm.at[pl.ds(row0, ROWS)],
                            acc.at[pl.ds(row0, ROWS)])
            plsc.subcore_barrier()

            @pl.loop(0, NB // 8)
            def _(g):
                descs = [
                    pltpu.async_copy(obuf, acc.at[idx.at[g * 8 + b]], sem,
                                     add=True)
                    for b in range(8)
                ]
                for d in descs:
                    d.wait()

            plsc.subcore_barrier()
            pltpu.sync_copy(acc.at[pl.ds(row0, ROWS)],
                            out_hbm.at[c, kind, pl.ds(row0, ROWS)])
            if kind == 0:
                plsc.subcore_barrier()

    return deg


K_BUF = 4
BA = 64
NBA = EP // (NC * NS) // BA


def _make_agg(C, FC):
    mesh = plsc.VectorSubcoreMesh(core_axis_name="c", subcore_axis_name="s")

    @functools.partial(
        pl.kernel,
        out_type=jax.ShapeDtypeStruct((NC, C, NP, FC), jnp.float32),
        mesh=mesh,
        scratch_types=(
            [pltpu.VMEM((NB, B), jnp.int32),
             pltpu.VMEM((NB, B), jnp.int32)]
            + [pltpu.VMEM((BA, FC), jnp.float32) for _ in range(K_BUF)]
            + [pltpu.VMEM_SHARED((NP, FC), jnp.float32)]
            + [pltpu.SemaphoreType.DMA for _ in range(2 * K_BUF)]
        ),
    )
    def agg(y_hbm, src_hbm, dst_hbm, z_hbm, out_hbm, idx_s, idx_d, *rest):
        gbufs = rest[:K_BUF]
        acc = rest[K_BUF]
        gsems = rest[K_BUF + 1:2 * K_BUF + 1]
        ssems = rest[2 * K_BUF + 1:]
        c = lax.axis_index("c")
        s = lax.axis_index("s")
        tid = c * NS + s
        pltpu.sync_copy(src_hbm.at[tid], idx_s)
        pltpu.sync_copy(dst_hbm.at[tid], idx_d)

        row0 = s * ROWS

        def fire_gather(ci, row, par, b):
            return pltpu.async_copy(
                y_hbm.at[ci].at[idx_s.at[row, pl.ds(par * BA, BA)]],
                gbufs[b], gsems[b])

        def wait_gather(ci, row, par, b):
            pltpu.make_async_copy(
                y_hbm.at[ci].at[idx_s.at[row, pl.ds(par * BA, BA)]],
                gbufs[b], gsems[b]).wait()

        def fire_scatter(row, par, b):
            return pltpu.async_copy(
                gbufs[b], acc.at[idx_d.at[row, pl.ds(par * BA, BA)]],
                ssems[b], add=True)

        def wait_scatter(row, par, b):
            pltpu.make_async_copy(
                gbufs[b], acc.at[idx_d.at[row, pl.ds(par * BA, BA)]],
                ssems[b]).wait()

        for ci in range(C):
            pltpu.sync_copy(z_hbm.at[pl.ds(row0, ROWS)],
                            acc.at[pl.ds(row0, ROWS)])
            plsc.subcore_barrier()

            fire_gather(ci, 0, 0, 0)
            fire_gather(ci, 0, 1, 1)
            for j in (0, 1):
                wait_gather(ci, j // 2, j % 2, j % K_BUF)
                fire_scatter(j // 2, j % 2, j % K_BUF)
                fire_gather(ci, (j + 2) // 2, j % 2, (j + 2) % K_BUF)

            @pl.loop(0, (NBA - 4) // K_BUF)
            def _(g):
                for b in range(K_BUF):
                    par = b % 2
                    wait_gather(ci, 2 * g + (2 + b) // 2, par,
                                (2 + b) % K_BUF)
                    fire_scatter(2 * g + (2 + b) // 2, par,
                                 (2 + b) % K_BUF)
                    wait_scatter(2 * g + (b // 2), par, b)
                    fire_gather(ci, 2 * g + 2 + (b // 2), par, b)

            for j in (NBA - 2, NBA - 1):
                wait_gather(ci, j // 2, j % 2, j % K_BUF)
                fire_scatter(j // 2, j % 2, j % K_BUF)
                wait_scatter((j - 2) // 2, j % 2, (j - 2) % K_BUF)
            for j in (NBA - 2, NBA - 1):
                wait_scatter(j // 2, j % 2, j % K_BUF)

            plsc.subcore_barrier()
            pltpu.sync_copy(acc.at[pl.ds(row0, ROWS)],
                            out_hbm.at[c, ci, pl.ds(row0, ROWS)])
            if ci + 1 < C:
                plsc.subcore_barrier()

    return agg


def _rs(deg_ref, kind):
    d = deg_ref[0, kind, :, 0:1] + deg_ref[1, kind, :, 0:1]
    return lax.rsqrt(jnp.maximum(d, 1.0))


def _mm1_body(x_ref, deg_ref, w_ref, o_ref):
    rout = _rs(deg_ref, 0)
    o_ref[0] = jnp.dot(x_ref[...] * rout, w_ref[...],
                       preferred_element_type=jnp.float32)


def _mm_mid_body(p_ref, deg_ref, b_ref, w_ref, o_ref, *, cin, fout):
    rin = _rs(deg_ref, 1)
    rout = _rs(deg_ref, 0)
    acc = jnp.zeros((MB, fout), jnp.float32)
    for k in range(cin):
        h = jnp.maximum((p_ref[0, k] + p_ref[1, k]) * rin + b_ref[k], 0.0)
        acc += jnp.dot(h * rout, w_ref[k], preferred_element_type=jnp.float32)
    if len(o_ref.shape) == 3:
        o_ref[0] = acc
    else:
        o_ref[...] = acc


def _fin_body(p_ref, deg_ref, b_ref, o_ref):
    rin = _rs(deg_ref, 1)
    o_ref[...] = (p_ref[0, 0, :, :64] + p_ref[1, 0, :, :64]) * rin + b_ref[0]


def _mm1(xp, deg, W1):
    return pl.pallas_call(
        _mm1_body,
        grid=(NP // MB, 4),
        in_specs=[
            pl.BlockSpec((MB, 256), lambda m, c: (m, 0)),
            pl.BlockSpec((2, 2, MB, 128), lambda m, c: (0, 0, m, 0)),
            pl.BlockSpec((256, 128), lambda m, c: (0, c)),
        ],
        out_specs=pl.BlockSpec((1, MB, 128), lambda m, c: (c, m, 0)),
        out_shape=jax.ShapeDtypeStruct((4, NP, 128), jnp.float32),
    )(xp, deg, W1)


def _mm2(p1, deg, b1r, W2r):
    return pl.pallas_call(
        functools.partial(_mm_mid_body, cin=4, fout=128),
        grid=(NP // MB, 4),
        in_specs=[
            pl.BlockSpec((2, 4, MB, 128), lambda m, c: (0, 0, m, 0)),
            pl.BlockSpec((2, 2, MB, 128), lambda m, c: (0, 0, m, 0)),
            pl.BlockSpec((4, 128), lambda m, c: (0, 0)),
            pl.BlockSpec((4, 128, 128), lambda m, c: (0, 0, c)),
        ],
        out_specs=pl.BlockSpec((1, MB, 128), lambda m, c: (c, m, 0)),
        out_shape=jax.ShapeDtypeStruct((4, NP, 128), jnp.float32),
    )(p1, deg, b1r, W2r)


def _mm3(p2, deg, b2r, W3r):
    return pl.pallas_call(
        functools.partial(_mm_mid_body, cin=4, fout=128),
        grid=(NP // MB,),
        in_specs=[
            pl.BlockSpec((2, 4, MB, 128), lambda m: (0, 0, m, 0)),
            pl.BlockSpec((2, 2, MB, 128), lambda m: (0, 0, m, 0)),
            pl.BlockSpec((4, 128), lambda m: (0, 0)),
            pl.BlockSpec((4, 128, 128), lambda m: (0, 0, 0)),
        ],
        out_specs=pl.BlockSpec((MB, 128), lambda m: (m, 0)),
        out_shape=jax.ShapeDtypeStruct((NP, 128), jnp.float32),
    )(p2, deg, b2r, W3r)


def _fin(p3, deg, b3r):
    return pl.pallas_call(
        _fin_body,
        grid=(NP // MB,),
        in_specs=[
            pl.BlockSpec((2, 1, MB, 128), lambda m: (0, 0, m, 0)),
            pl.BlockSpec((2, 2, MB, 128), lambda m: (0, 0, m, 0)),
            pl.BlockSpec((1, 64), lambda m: (0, 0)),
        ],
        out_specs=pl.BlockSpec((MB, 64), lambda m: (m, 0)),
        out_shape=jax.ShapeDtypeStruct((NP, 64), jnp.float32),
    )(p3, deg, b3r)


_deg_kernel = _make_deg()
_agg4 = _make_agg(4, 128)
_agg1 = _make_agg(1, 128)


def kernel(x, edge_index, W1, b1, W2, b2, W3, b3):
    src = edge_index[0]
    dst = edge_index[1]
    npad = EP - E
    pad_idx = N + (jnp.arange(npad, dtype=jnp.int32) % (NP - N))
    src1 = jnp.concatenate([src, pad_idx])
    dst1 = jnp.concatenate([dst, pad_idx])
    srcp = src1.reshape(NC * NS, NB, B)
    dstp = dst1.reshape(NC * NS, NB, B)
    srca, dsta = srcp, dstp
    xp = jnp.pad(x, ((0, NP - N), (0, 0)))

    zeros = jnp.zeros((NP, 128), jnp.float32)
    deg = _deg_kernel(srcp, dstp, zeros)
    y1 = _mm1(xp, deg, W1)
    p1 = _agg4(y1, srca, dsta, zeros)
    y2 = _mm2(p1, deg, b1.reshape(4, 128), W2.reshape(4, 128, 512))
    p2 = _agg4(y2, srca, dsta, zeros)
    W3p = jnp.pad(W3.reshape(4, 128, 64), ((0, 0), (0, 0), (0, 64)))
    y3 = _mm3(p2, deg, b2.reshape(4, 128), W3p)
    p3 = _agg1(y3.reshape(1, NP, 128), srca, dsta, zeros)
    out = _fin(p3, deg, b3.reshape(1, 64))
    return out[:N]


# speedup vs baseline: 5.0477x; 1.0675x over previous
import functools

import jax
import jax.numpy as jnp
from jax import lax
from jax.experimental import pallas as pl
from jax.experimental.pallas import tpu as pltpu
from jax.experimental.pallas import tpu_sc as plsc

N = 10000
NP = 10240
E = 160000
EP = 163840
NB = 40
B = 128
NC = 2
NS = 16
ROWS = NP // NS
MB = 512


def _make_deg():
    mesh = plsc.VectorSubcoreMesh(core_axis_name="c", subcore_axis_name="s")

    @functools.partial(
        pl.kernel,
        out_type=jax.ShapeDtypeStruct((NC, 2, NP, 128), jnp.float32),
        mesh=mesh,
        scratch_types=[
            pltpu.VMEM((NB, B), jnp.int32),
            pltpu.VMEM((NB, B), jnp.int32),
            pltpu.VMEM((B, 128), jnp.float32),
            pltpu.VMEM_SHARED((NP, 128), jnp.float32),
            pltpu.SemaphoreType.DMA,
        ],
    )
    def deg(src_hbm, dst_hbm, z_hbm, out_hbm, idx_s, idx_d, obuf, acc, sem):
        c = lax.axis_index("c")
        s = lax.axis_index("s")
        tid = c * NS + s
        pltpu.sync_copy(src_hbm.at[tid], idx_s)
        pltpu.sync_copy(dst_hbm.at[tid], idx_d)

        @pl.loop(0, B)
        def _(i):
            for k in range(8):
                obuf[i, pl.ds(k * 16, 16)] = jnp.ones((16,), jnp.float32)

        row0 = s * ROWS
        for kind, idx in ((0, idx_s), (1, idx_d)):
            pltpu.sync_copy(z_hbm.at[pl.ds(row0, ROWS)],
                            acc.at[pl.ds(row0, ROWS)])
            plsc.subcore_barrier()

            @pl.loop(0, NB // 8)
            def _(g):
                descs = [
                    pltpu.async_copy(obuf, acc.at[idx.at[g * 8 + b]], sem,
                                     add=True)
                    for b in range(8)
                ]
                for d in descs:
                    d.wait()

            plsc.subcore_barrier()
            pltpu.sync_copy(acc.at[pl.ds(row0, ROWS)],
                            out_hbm.at[c, kind, pl.ds(row0, ROWS)])
            if kind == 0:
                plsc.subcore_barrier()

    return deg


K_BUF = 8
HID = 4
BA = 32
NBA = EP // (NC * NS) // BA
PERROW = B // BA


def _make_agg(C, FC):
    mesh = plsc.VectorSubcoreMesh(core_axis_name="c", subcore_axis_name="s")

    @functools.partial(
        pl.kernel,
        out_type=jax.ShapeDtypeStruct((NC, C, NP, FC), jnp.float32),
        mesh=mesh,
        scratch_types=(
            [pltpu.VMEM((NB, B), jnp.int32),
             pltpu.VMEM((NB, B), jnp.int32)]
            + [pltpu.VMEM((BA, FC), jnp.float32) for _ in range(K_BUF)]
            + [pltpu.VMEM_SHARED((NP, FC), jnp.float32)]
            + [pltpu.SemaphoreType.DMA for _ in range(2 * K_BUF)]
        ),
    )
    def agg(y_hbm, src_hbm, dst_hbm, z_hbm, out_hbm, idx_s, idx_d, *rest):
        gbufs = rest[:K_BUF]
        acc = rest[K_BUF]
        gsems = rest[K_BUF + 1:2 * K_BUF + 1]
        ssems = rest[2 * K_BUF + 1:]
        c = lax.axis_index("c")
        s = lax.axis_index("s")
        tid = c * NS + s
        pltpu.sync_copy(src_hbm.at[tid], idx_s)
        pltpu.sync_copy(dst_hbm.at[tid], idx_d)

        row0 = s * ROWS

        def fire_gather(ci, row, par, b):
            return pltpu.async_copy(
                y_hbm.at[ci].at[idx_s.at[row, pl.ds(par * BA, BA)]],
                gbufs[b], gsems[b])

        def wait_gather(ci, row, par, b):
            pltpu.make_async_copy(
                y_hbm.at[ci].at[idx_s.at[row, pl.ds(par * BA, BA)]],
                gbufs[b], gsems[b]).wait()

        def fire_scatter(row, par, b):
            return pltpu.async_copy(
                gbufs[b], acc.at[idx_d.at[row, pl.ds(par * BA, BA)]],
                ssems[b], add=True)

        def wait_scatter(row, par, b):
            pltpu.make_async_copy(
                gbufs[b], acc.at[idx_d.at[row, pl.ds(par * BA, BA)]],
                ssems[b]).wait()

        for ci in range(C):
            pltpu.sync_copy(z_hbm.at[pl.ds(row0, ROWS)],
                            acc.at[pl.ds(row0, ROWS)])
            plsc.subcore_barrier()

            for j in range(HID):
                fire_gather(ci, j // PERROW, j % PERROW, j % K_BUF)
            for j in range(HID):
                wait_gather(ci, j // PERROW, j % PERROW, j % K_BUF)
                fire_scatter(j // PERROW, j % PERROW, j % K_BUF)
                fire_gather(ci, (j + HID) // PERROW, (j + HID) % PERROW,
                            (j + HID) % K_BUF)

            @pl.loop(0, (NBA - 2 * HID) // K_BUF)
            def _(g):
                for b in range(K_BUF):
                    par = b % PERROW
                    row = 2 * g + 1 + (b // PERROW)
                    wait_gather(ci, row, par, (HID + b) % K_BUF)
                    fire_scatter(row, par, (HID + b) % K_BUF)
                    wait_scatter(row - 1, par, b)
                    fire_gather(ci, row + 1, par, b)

            for j in range(NBA - HID, NBA):
                wait_gather(ci, j // PERROW, j % PERROW, j % K_BUF)
                fire_scatter(j // PERROW, j % PERROW, j % K_BUF)
                wait_scatter((j - HID) // PERROW, (j - HID) % PERROW,
                             (j - HID) % K_BUF)
            for j in range(NBA - HID, NBA):
                wait_scatter(j // PERROW, j % PERROW, j % K_BUF)

            plsc.subcore_barrier()
            pltpu.sync_copy(acc.at[pl.ds(row0, ROWS)],
                            out_hbm.at[c, ci, pl.ds(row0, ROWS)])

    return agg


def _rs(deg_ref, kind):
    d = deg_ref[0, kind, :, 0:1] + deg_ref[1, kind, :, 0:1]
    return lax.rsqrt(jnp.maximum(d, 1.0))


def _mm1_body(x_ref, deg_ref, w_ref, o_ref):
    rout = _rs(deg_ref, 0)
    o_ref[0] = jnp.dot(x_ref[...] * rout, w_ref[...],
                       preferred_element_type=jnp.float32)


def _mm_mid_body(p_ref, deg_ref, b_ref, w_ref, o_ref, *, cin, fout):
    rin = _rs(deg_ref, 1)
    rout = _rs(deg_ref, 0)
    acc = jnp.zeros((MB, fout), jnp.float32)
    for k in range(cin):
        h = jnp.maximum((p_ref[0, k] + p_ref[1, k]) * rin + b_ref[k], 0.0)
        acc += jnp.dot(h * rout, w_ref[k], preferred_element_type=jnp.float32)
    if len(o_ref.shape) == 3:
        o_ref[0] = acc
    else:
        o_ref[...] = acc


def _fin_body(p_ref, deg_ref, b_ref, o_ref):
    rin = _rs(deg_ref, 1)
    o_ref[...] = (p_ref[0, 0, :, :64] + p_ref[1, 0, :, :64]) * rin + b_ref[0]


def _mm1(xp, deg, W1):
    return pl.pallas_call(
        _mm1_body,
        grid=(NP // MB, 4),
        in_specs=[
            pl.BlockSpec((MB, 256), lambda m, c: (m, 0)),
            pl.BlockSpec((2, 2, MB, 128), lambda m, c: (0, 0, m, 0)),
            pl.BlockSpec((256, 128), lambda m, c: (0, c)),
        ],
        out_specs=pl.BlockSpec((1, MB, 128), lambda m, c: (c, m, 0)),
        out_shape=jax.ShapeDtypeStruct((4, NP, 128), jnp.float32),
    )(xp, deg, W1)


def _mm2(p1, deg, b1r, W2r):
    return pl.pallas_call(
        functools.partial(_mm_mid_body, cin=4, fout=128),
        grid=(NP // MB, 4),
        in_specs=[
            pl.BlockSpec((2, 4, MB, 128), lambda m, c: (0, 0, m, 0)),
            pl.BlockSpec((2, 2, MB, 128), lambda m, c: (0, 0, m, 0)),
            pl.BlockSpec((4, 128), lambda m, c: (0, 0)),
            pl.BlockSpec((4, 128, 128), lambda m, c: (0, 0, c)),
        ],
        out_specs=pl.BlockSpec((1, MB, 128), lambda m, c: (c, m, 0)),
        out_shape=jax.ShapeDtypeStruct((4, NP, 128), jnp.float32),
    )(p1, deg, b1r, W2r)


def _mm3(p2, deg, b2r, W3r):
    return pl.pallas_call(
        functools.partial(_mm_mid_body, cin=4, fout=128),
        grid=(NP // MB,),
        in_specs=[
            pl.BlockSpec((2, 4, MB, 128), lambda m: (0, 0, m, 0)),
            pl.BlockSpec((2, 2, MB, 128), lambda m: (0, 0, m, 0)),
            pl.BlockSpec((4, 128), lambda m: (0, 0)),
            pl.BlockSpec((4, 128, 128), lambda m: (0, 0, 0)),
        ],
        out_specs=pl.BlockSpec((MB, 128), lambda m: (m, 0)),
        out_shape=jax.ShapeDtypeStruct((NP, 128), jnp.float32),
    )(p2, deg, b2r, W3r)


def _fin(p3, deg, b3r):
    return pl.pallas_call(
        _fin_body,
        grid=(NP // MB,),
        in_specs=[
            pl.BlockSpec((2, 1, MB, 128), lambda m: (0, 0, m, 0)),
            pl.BlockSpec((2, 2, MB, 128), lambda m: (0, 0, m, 0)),
            pl.BlockSpec((1, 64), lambda m: (0, 0)),
        ],
        out_specs=pl.BlockSpec((MB, 64), lambda m: (m, 0)),
        out_shape=jax.ShapeDtypeStruct((NP, 64), jnp.float32),
    )(p3, deg, b3r)


_deg_kernel = _make_deg()
_agg4 = _make_agg(4, 128)
_agg1 = _make_agg(1, 128)


def kernel(x, edge_index, W1, b1, W2, b2, W3, b3):
    src = edge_index[0]
    dst = edge_index[1]
    npad = EP - E
    pad_idx = N + (jnp.arange(npad, dtype=jnp.int32) % (NP - N))
    src1 = jnp.concatenate([src, pad_idx])
    dst1 = jnp.concatenate([dst, pad_idx])
    srcp = src1.reshape(NC * NS, NB, B)
    dstp = dst1.reshape(NC * NS, NB, B)
    srca, dsta = srcp, dstp
    xp = jnp.pad(x, ((0, NP - N), (0, 0)))

    zeros = jnp.zeros((NP, 128), jnp.float32)
    deg = _deg_kernel(srcp, dstp, zeros)
    y1 = _mm1(xp, deg, W1)
    p1 = _agg4(y1, srca, dsta, zeros)
    y2 = _mm2(p1, deg, b1.reshape(4, 128), W2.reshape(4, 128, 512))
    p2 = _agg4(y2, srca, dsta, zeros)
    W3p = jnp.pad(W3.reshape(4, 128, 64), ((0, 0), (0, 0), (0, 64)))
    y3 = _mm3(p2, deg, b2.reshape(4, 128), W3p)
    p3 = _agg1(y3.reshape(1, NP, 128), srca, dsta, zeros)
    out = _fin(p3, deg, b3.reshape(1, 64))
    return out[:N]
